# baseline (speedup 1.0000x reference)
import functools

import jax
import jax.numpy as jnp
from jax.experimental import pallas as pl
from jax.experimental.pallas import tpu as pltpu

_ALPHA = 0.25
_OHEM_RATIO = 0.7


def _fused_body(tgt_ref, xt_ref, out_ref, facc_ref, *, n_classes, blk,
                n_blocks, k):
    i = pl.program_id(0)
    t = tgt_ref[0, 0, :]
    n_chunks = n_classes // 8
    ridx0 = jax.lax.broadcasted_iota(jnp.int32, (8, blk), 0)
    t_b = jnp.broadcast_to(t[None, :], (8, blk))

    def step_a(c, carry):
        acc_m, acc_t = carry
        xc = xt_ref[pl.ds(c * 8, 8), :]
        hit = (ridx0 + c * 8) == t_b
        return (jnp.maximum(acc_m, xc),
                acc_t + jnp.where(hit, xc, 0.0))

    acc_m, acc_t = jax.lax.fori_loop(
        0, n_chunks, step_a,
        (jnp.full((8, blk), -jnp.inf, jnp.float32),
         jnp.zeros((8, blk), jnp.float32)))
    m = jnp.max(acc_m, axis=0)
    tl = jnp.sum(acc_t, axis=0)
    m_b = jnp.broadcast_to(m[None, :], (8, blk))

    def step_b(c, acc_s):
        xc = xt_ref[pl.ds(c * 8, 8), :]
        return acc_s + jnp.exp(xc - m_b)

    acc_s = jax.lax.fori_loop(0, n_chunks, step_b,
                              jnp.zeros((8, blk), jnp.float32))
    s = jnp.sum(acc_s, axis=0)
    lse = m + jnp.log(s)
    ce = lse - tl
    pt = jnp.exp(-ce)
    one_m = 1.0 - pt
    f = _ALPHA * one_m * one_m * ce
    facc_ref[pl.ds(i, 1), :] = f[None, :]

    @pl.when(i == n_blocks - 1)
    def _select():
        fall = facc_ref[...]
        bits = jax.lax.bitcast_convert_type(fall, jnp.int32)

        def step(j, prefix):
            cand = prefix | (jnp.int32(1) << (jnp.int32(30) - j))
            cnt = jnp.sum((bits >= cand).astype(jnp.int32))
            return jnp.where(cnt >= k, cand, prefix)

        thr = jax.lax.fori_loop(0, 31, step, jnp.int32(0))
        gt = bits > thr
        cnt_gt = jnp.sum(gt.astype(jnp.int32))
        sum_gt = jnp.sum(jnp.where(gt, fall, 0.0))
        thr_f = jnp.max(jnp.where(bits == thr, fall, 0.0))
        res = (
            sum_gt + (jnp.int32(k) - cnt_gt).astype(jnp.float32) * thr_f
        ) / jnp.float32(k)
        out_ref[...] = res[None, None]


def kernel(inputs, targets):
    n, c = inputs.shape
    blk = 1024
    g = n // blk
    k = int(_OHEM_RATIO * n)
    xt = inputs.T
    tgt = targets.astype(jnp.int32).reshape(g, 1, blk)
    body = functools.partial(_fused_body, n_classes=c, blk=blk, n_blocks=g,
                             k=k)
    out = pl.pallas_call(
        body,
        grid=(g,),
        in_specs=[
            pl.BlockSpec((1, 1, blk), lambda i: (i, 0, 0)),
            pl.BlockSpec((c, blk), lambda i: (0, i)),
        ],
        out_specs=pl.BlockSpec((1, 1), lambda i: (0, 0)),
        out_shape=jax.ShapeDtypeStruct((1, 1), jnp.float32),
        scratch_shapes=[pltpu.VMEM((g, blk), jnp.float32)],
    )(tgt, xt)
    return out[0, 0]


# speedup vs baseline: 1.7677x; 1.7677x over previous
import functools

import jax
import jax.numpy as jnp
from jax.experimental import pallas as pl
from jax.experimental.pallas import tpu as pltpu

_ALPHA = 0.25
_OHEM_RATIO = 0.7


def _fused_body(tgt_ref, xt_ref, out_ref, facc_ref, *, n_classes, blk,
                n_blocks, k):
    i = pl.program_id(0)
    t = tgt_ref[0, 0, :]
    n_chunks = n_classes // 8
    ridx0 = jax.lax.broadcasted_iota(jnp.int32, (8, blk), 0)
    t_b = jnp.broadcast_to(t[None, :], (8, blk))

    acc_m = jnp.full((8, blk), -jnp.inf, jnp.float32)
    acc_t = jnp.zeros((8, blk), jnp.float32)
    for c in range(n_chunks):
        xc = xt_ref[c * 8:(c + 1) * 8, :]
        hit = (ridx0 + c * 8) == t_b
        acc_m = jnp.maximum(acc_m, xc)
        acc_t = acc_t + jnp.where(hit, xc, 0.0)
    m = jnp.max(acc_m, axis=0)
    tl = jnp.sum(acc_t, axis=0)
    m_b = jnp.broadcast_to(m[None, :], (8, blk))

    acc_s = jnp.zeros((8, blk), jnp.float32)
    for c in range(n_chunks):
        xc = xt_ref[c * 8:(c + 1) * 8, :]
        acc_s = acc_s + jnp.exp(xc - m_b)
    s = jnp.sum(acc_s, axis=0)
    lse = m + jnp.log(s)
    ce = lse - tl
    pt = jnp.exp(-ce)
    one_m = 1.0 - pt
    f = _ALPHA * one_m * one_m * ce
    facc_ref[pl.ds(i, 1), :] = f[None, :]

    @pl.when(i == n_blocks - 1)
    def _select():
        fall = facc_ref[...]
        bits = jax.lax.bitcast_convert_type(fall, jnp.int32)

        def step(j, prefix):
            cand = prefix | (jnp.int32(1) << (jnp.int32(30) - j))
            cnt = jnp.sum((bits >= cand).astype(jnp.int32))
            return jnp.where(cnt >= k, cand, prefix)

        thr = jax.lax.fori_loop(0, 31, step, jnp.int32(0))
        gt = bits > thr
        cnt_gt = jnp.sum(gt.astype(jnp.int32))
        sum_gt = jnp.sum(jnp.where(gt, fall, 0.0))
        thr_f = jnp.max(jnp.where(bits == thr, fall, 0.0))
        res = (
            sum_gt + (jnp.int32(k) - cnt_gt).astype(jnp.float32) * thr_f
        ) / jnp.float32(k)
        out_ref[...] = res[None, None]


def kernel(inputs, targets):
    n, c = inputs.shape
    blk = 1024
    g = n // blk
    k = int(_OHEM_RATIO * n)
    xt = inputs.T
    tgt = targets.astype(jnp.int32).reshape(g, 1, blk)
    body = functools.partial(_fused_body, n_classes=c, blk=blk, n_blocks=g,
                             k=k)
    out = pl.pallas_call(
        body,
        grid=(g,),
        in_specs=[
            pl.BlockSpec((1, 1, blk), lambda i: (i, 0, 0)),
            pl.BlockSpec((c, blk), lambda i: (0, i)),
        ],
        out_specs=pl.BlockSpec((1, 1), lambda i: (0, 0)),
        out_shape=jax.ShapeDtypeStruct((1, 1), jnp.float32),
        scratch_shapes=[pltpu.VMEM((g, blk), jnp.float32)],
    )(tgt, xt)
    return out[0, 0]


# speedup vs baseline: 1.9410x; 1.0980x over previous
import functools

import jax
import jax.numpy as jnp
from jax.experimental import pallas as pl
from jax.experimental.pallas import tpu as pltpu

_ALPHA = 0.25
_OHEM_RATIO = 0.7


def _fused_body(tgt_ref, xt_ref, out_ref, facc_ref, *, n_classes, blk,
                n_blocks, k):
    i = pl.program_id(0)
    t = tgt_ref[0, 0, :]
    n_chunks = n_classes // 8
    ridx0 = jax.lax.broadcasted_iota(jnp.int32, (8, blk), 0)
    t_b = jnp.broadcast_to(t[None, :], (8, blk))

    acc_m = jnp.full((8, blk), -jnp.inf, jnp.float32)
    for c in range(n_chunks):
        acc_m = jnp.maximum(acc_m, xt_ref[c * 8:(c + 1) * 8, :])
    m = jnp.max(acc_m, axis=0)
    m_b = jnp.broadcast_to(m[None, :], (8, blk))

    acc_s = jnp.zeros((8, blk), jnp.float32)
    for c in range(n_chunks):
        acc_s = acc_s + jnp.exp(xt_ref[c * 8:(c + 1) * 8, :] - m_b)
    s = jnp.sum(acc_s, axis=0)

    acc_t = jnp.zeros((8, blk), jnp.float32)
    for c in range(n_chunks):
        xc = xt_ref[c * 8:(c + 1) * 8, :]
        hit = (ridx0 + c * 8) == t_b
        acc_t = acc_t + jnp.where(hit, xc, 0.0)
    tl = jnp.sum(acc_t, axis=0)
    lse = m + jnp.log(s)
    ce = lse - tl
    pt = jnp.exp(-ce)
    one_m = 1.0 - pt
    f = _ALPHA * one_m * one_m * ce
    facc_ref[pl.ds(i, 1), :] = f[None, :]

    @pl.when(i == n_blocks - 1)
    def _select():
        fall = facc_ref[...]
        bits = jax.lax.bitcast_convert_type(fall, jnp.int32)

        def step(j, prefix):
            cand = prefix | (jnp.int32(1) << (jnp.int32(30) - j))
            cnt = jnp.sum((bits >= cand).astype(jnp.int32))
            return jnp.where(cnt >= k, cand, prefix)

        thr = jax.lax.fori_loop(0, 31, step, jnp.int32(0))
        gt = bits > thr
        cnt_gt = jnp.sum(gt.astype(jnp.int32))
        sum_gt = jnp.sum(jnp.where(gt, fall, 0.0))
        thr_f = jnp.max(jnp.where(bits == thr, fall, 0.0))
        res = (
            sum_gt + (jnp.int32(k) - cnt_gt).astype(jnp.float32) * thr_f
        ) / jnp.float32(k)
        out_ref[...] = res[None, None]


def kernel(inputs, targets):
    n, c = inputs.shape
    blk = 2048
    g = n // blk
    k = int(_OHEM_RATIO * n)
    xt = inputs.T
    tgt = targets.astype(jnp.int32).reshape(g, 1, blk)
    body = functools.partial(_fused_body, n_classes=c, blk=blk, n_blocks=g,
                             k=k)
    out = pl.pallas_call(
        body,
        grid=(g,),
        in_specs=[
            pl.BlockSpec((1, 1, blk), lambda i: (i, 0, 0)),
            pl.BlockSpec((c, blk), lambda i: (0, i)),
        ],
        out_specs=pl.BlockSpec((1, 1), lambda i: (0, 0)),
        out_shape=jax.ShapeDtypeStruct((1, 1), jnp.float32),
        scratch_shapes=[pltpu.VMEM((g, blk), jnp.float32)],
    )(tgt, xt)
    return out[0, 0]
